# Initial kernel scaffold; baseline (speedup 1.0000x reference)
import jax
import jax.numpy as jnp
from jax.experimental import pallas as pl


def kernel(h, edge_index):
    raise NotImplementedError("write your pallas kernel here")


# speedup vs baseline: 1.2065x; 1.2065x over previous
import jax
import jax.numpy as jnp
from jax import lax
from jax.experimental import pallas as pl
from jax.experimental.pallas import tpu as pltpu
from jax.experimental.pallas import tpu_sc as plsc

N_NODES = 10000
D_FEAT = 128
N_EDGES = 320000

_NC = 2
_NS = 16
_L = 16
_NW = _NC * _NS
_PER_TILE = N_EDGES // _NW
_CH = 400
_NCHUNK = _PER_TILE // _CH
_SG = 80
_NSG = _CH // _SG
_NG = _CH // _L


def _dot_body(h_hbm, src_hbm, dst_hbm, out_hbm,
              idx_u, idx_v, rows_u, rows_v, out_c, sem):
    wid = lax.axis_index("c") * _NS + lax.axis_index("s")
    base0 = wid * _PER_TILE
    lanes = lax.iota(jnp.int32, _L)

    def chunk_body(i, carry):
        base = base0 + i * _CH
        pltpu.sync_copy(src_hbm.at[pl.ds(base, _CH)], idx_u)
        pltpu.sync_copy(dst_hbm.at[pl.ds(base, _CH)], idx_v)
        cps = []
        for j in range(_NSG):
            sl = pl.ds(j * _SG, _SG)
            cps.append(pltpu.async_copy(h_hbm.at[idx_u.at[sl]], rows_u.at[sl], sem))
            cps.append(pltpu.async_copy(h_hbm.at[idx_v.at[sl]], rows_v.at[sl], sem))
        for cp in cps:
            cp.wait()

        def group_body(g, gcarry):
            rid = g * _L + lanes
            acc = jnp.zeros((_L,), jnp.float32)
            for d in range(D_FEAT):
                dcol = jnp.full((_L,), d, jnp.int32)
                u = plsc.load_gather(rows_u, [rid, dcol])
                v = plsc.load_gather(rows_v, [rid, dcol])
                acc = acc + u * v
            out_c[pl.ds(g * _L, _L)] = acc
            return gcarry

        lax.fori_loop(0, _NG, group_body, 0)
        pltpu.sync_copy(out_c, out_hbm.at[pl.ds(base, _CH)])
        return carry

    lax.fori_loop(0, _NCHUNK, chunk_body, 0)


@jax.jit
def kernel(h, edge_index):
    src = edge_index[0]
    dst = edge_index[1]
    mesh = plsc.VectorSubcoreMesh(
        core_axis_name="c", subcore_axis_name="s",
        num_cores=_NC, num_subcores=_NS)
    f = pl.kernel(
        _dot_body,
        out_type=jax.ShapeDtypeStruct((N_EDGES,), jnp.float32),
        mesh=mesh,
        scratch_types=[
            pltpu.VMEM((_CH,), jnp.int32),
            pltpu.VMEM((_CH,), jnp.int32),
            pltpu.VMEM((_CH, D_FEAT), jnp.float32),
            pltpu.VMEM((_CH, D_FEAT), jnp.float32),
            pltpu.VMEM((_CH,), jnp.float32),
            pltpu.SemaphoreType.DMA,
        ],
        compiler_params=pltpu.CompilerParams(needs_layout_passes=False),
    )
    return f(h, src, dst)
